# baseline (speedup 1.0000x reference)
import functools

import jax
import jax.numpy as jnp
from jax.experimental import pallas as pl

_T = 2560
_K = 128


def _fused_kernel(ids_ref, x_ref, w_ref, b_ref, out_ref, *, n_atoms):
    i = pl.program_id(0)

    @pl.when(i == 0)
    def _init():
        out_ref[...] = jnp.zeros_like(out_ref)

    x = x_ref[...].astype(jnp.bfloat16)
    w = w_ref[...].astype(jnp.bfloat16)
    y = jax.lax.dot_general(
        x, w, (((1,), (1,)), ((), ())), preferred_element_type=jnp.float32
    )
    y = jnp.maximum(y + b_ref[...], 0.0).astype(jnp.bfloat16)

    ids = ids_ref[0]
    t = ids.shape[1]
    first = ids[0, 0]
    last = ids[0, t - 1]
    sentinel = jnp.int32(n_atoms)

    def cond(base):
        return base <= last

    def body(base):
        cur = jnp.minimum(base, jnp.int32(n_atoms - _K))
        cur = (cur // 8) * 8
        row = jax.lax.broadcasted_iota(jnp.int32, (_K, t), 0) + cur
        onehot = jnp.where((row == ids) & (ids >= base), 1.0, 0.0).astype(
            jnp.bfloat16
        )
        partial = jax.lax.dot_general(
            onehot, y, (((1,), (0,)), ((), ())), preferred_element_type=jnp.float32
        )
        out_ref[pl.ds(cur, _K), :] += partial
        nxt = jnp.min(jnp.where(ids >= cur + _K, ids, sentinel))
        return nxt

    jax.lax.while_loop(cond, body, first)


def kernel(pair_features, pair_split, W, b):
    n_pairs, in_feats = pair_features.shape
    out_feats = W.shape[0]
    n_atoms = 10000
    grid = n_pairs // _T
    ids3 = pair_split.reshape(grid, 1, _T)
    b2 = b.reshape(1, out_feats)
    return pl.pallas_call(
        functools.partial(_fused_kernel, n_atoms=n_atoms),
        grid=(grid,),
        in_specs=[
            pl.BlockSpec((1, 1, _T), lambda i: (i, 0, 0)),
            pl.BlockSpec((_T, in_feats), lambda i: (i, 0)),
            pl.BlockSpec((out_feats, in_feats), lambda i: (0, 0)),
            pl.BlockSpec((1, out_feats), lambda i: (0, 0)),
        ],
        out_specs=pl.BlockSpec((n_atoms, out_feats), lambda i: (0, 0)),
        out_shape=jax.ShapeDtypeStruct((n_atoms, out_feats), jnp.float32),
    )(ids3, pair_features, W, b2)


# speedup vs baseline: 1.0353x; 1.0353x over previous
import functools

import jax
import jax.numpy as jnp
from jax.experimental import pallas as pl

_T = 2560
_K = 128


def _fused_kernel(ids_ref, x_ref, w_ref, b_ref, out_ref, *, n_atoms):
    i = pl.program_id(0)

    @pl.when(i == 0)
    def _init():
        out_ref[...] = jnp.zeros_like(out_ref)

    x = x_ref[...]
    w = w_ref[...]
    y = jax.lax.dot_general(
        x, w, (((1,), (1,)), ((), ())), preferred_element_type=jnp.float32
    )
    y = jnp.maximum(y + b_ref[...], 0.0)

    ids = ids_ref[0]
    t = ids.shape[1]
    first = ids[0, 0]
    last = ids[0, t - 1]
    sentinel = jnp.int32(n_atoms)

    def cond(base):
        return base <= last

    def body(base):
        cur = jnp.minimum(base, jnp.int32(n_atoms - _K))
        cur = (cur // 8) * 8
        row = jax.lax.broadcasted_iota(jnp.int32, (_K, t), 0) + cur
        onehot = jnp.where((row == ids) & (ids >= base), 1.0, 0.0)
        partial = jax.lax.dot_general(
            onehot, y, (((1,), (0,)), ((), ())), preferred_element_type=jnp.float32
        )
        out_ref[pl.ds(cur, _K), :] += partial
        nxt = jnp.min(jnp.where(ids >= cur + _K, ids, sentinel))
        return nxt

    jax.lax.while_loop(cond, body, first)


def kernel(pair_features, pair_split, W, b):
    n_pairs, in_feats = pair_features.shape
    out_feats = W.shape[0]
    n_atoms = 10000
    grid = n_pairs // _T
    ids3 = pair_split.reshape(grid, 1, _T)
    b2 = b.reshape(1, out_feats)
    return pl.pallas_call(
        functools.partial(_fused_kernel, n_atoms=n_atoms),
        grid=(grid,),
        in_specs=[
            pl.BlockSpec((1, 1, _T), lambda i: (i, 0, 0)),
            pl.BlockSpec((_T, in_feats), lambda i: (i, 0)),
            pl.BlockSpec((out_feats, in_feats), lambda i: (0, 0)),
            pl.BlockSpec((1, out_feats), lambda i: (0, 0)),
        ],
        out_specs=pl.BlockSpec((n_atoms, out_feats), lambda i: (0, 0)),
        out_shape=jax.ShapeDtypeStruct((n_atoms, out_feats), jnp.float32),
    )(ids3, pair_features, W, b2)
